# baseline (speedup 1.0000x reference)
import functools

import jax
import jax.numpy as jnp
from jax import lax
from jax.experimental import pallas as pl
from jax.experimental.pallas import tpu as pltpu
from jax.experimental.pallas import tpu_sc as plsc

N, D, K = 32768, 128, 1024
NT = 16
C = N // NT
CV = C // 16
CAP = 256
CAPV = CAP // 16
KV = K // NT
SHIFTS = (24, 16, 8, 0)
INT_MIN = -(2**31)


def _topk_perm(Xf):
    mesh = plsc.VectorSubcoreMesh(
        core_axis_name="c", subcore_axis_name="s", num_cores=1)

    @functools.partial(
        pl.kernel,
        out_type=jax.ShapeDtypeStruct((K, D), jnp.float32),
        mesh=mesh,
        compiler_params=pltpu.CompilerParams(needs_layout_passes=False),
        scratch_types=[
            pltpu.VMEM((C,), jnp.float32),
            pltpu.VMEM((C,), jnp.int32),
            pltpu.VMEM((C,), jnp.int32),
            pltpu.VMEM((256,), jnp.int32),
            pltpu.VMEM((256,), jnp.int32),
            pltpu.VMEM((256,), jnp.int32),
            pltpu.VMEM((NT * 256,), jnp.int32),
            pltpu.VMEM((16,), jnp.int32),
            pltpu.VMEM((NT * 16,), jnp.int32),
            pltpu.VMEM((CAP,), jnp.int32),
            pltpu.VMEM((CAP,), jnp.int32),
            pltpu.VMEM((CAP,), jnp.int32),
            pltpu.VMEM((CAP,), jnp.int32),
            pltpu.VMEM((CAP,), jnp.int32),
            pltpu.VMEM((KV,), jnp.int32),
            pltpu.VMEM((KV,), jnp.int32),
            pltpu.VMEM((KV,), jnp.int32),
            pltpu.VMEM((KV,), jnp.int32),
            pltpu.VMEM((KV, D), jnp.float32),
            pltpu.VMEM_SHARED((NT * 256,), jnp.int32),
            pltpu.VMEM_SHARED((NT * 16,), jnp.int32),
            pltpu.VMEM_SHARED((K + NT,), jnp.int32),
            pltpu.VMEM_SHARED((K + NT,), jnp.int32),
            pltpu.VMEM_SHARED((K + NT,), jnp.int32),
            pltpu.VMEM_SHARED((K + NT,), jnp.int32),
            pltpu.VMEM_SHARED((K + NT,), jnp.int32),
            pltpu.SemaphoreType.DMA,
        ],
    )
    def k(xf_hbm, out_hbm, colflat, idxcol, keys, bins, pret,
          offt, allhist, cntv,
          allcnt, gtkey, gtidx, gtdst, eqidx, eqdst, entk, enti, entd,
          myperm, rows, hist_sh, cnt_sh, akey_sh, aidx_sh, bkey_sh,
          bidx_sh, perm_sh, sem):
        tid = lax.axis_index("s")
        iota = lax.iota(jnp.int32, 16)
        zero16 = iota ^ iota

        sc1 = jax.named_scope("p1_col"); sc1.__enter__()
        def idx_body(j, _):
            for u in range(4):
                base = j * 64 + u * 16
                idxcol[pl.ds(base, 16)] = (tid * C + base + iota) * D + (
                    D - 1)
            return 0

        lax.fori_loop(0, CV // 4, idx_body, 0)
        col_copies = [
            pltpu.async_copy(
                xf_hbm.at[idxcol.at[pl.ds(i * 128, 128)]],
                colflat.at[pl.ds(i * 128, 128)], sem)
            for i in range(16)
        ]
        for cp in col_copies:
            cp.wait()

        sc1.__exit__(None, None, None)
        sc2 = jax.named_scope("p2_keys"); sc2.__enter__()
        def keys_body(j, _):
            for u in range(4):
                base = j * 64 + u * 16
                v = colflat[pl.ds(base, 16)]
                b = plsc.bitcast(v, jnp.int32)
                big = jnp.where(b >= 0, b, ~b ^ INT_MIN)
                big = jnp.where(b == INT_MIN, 0, big)
                keys[pl.ds(base, 16)] = big
            return 0

        lax.fori_loop(0, CV // 4, keys_body, 0)

        occ0, _ = plsc.scan_count(zero16)
        occ_base = jnp.max(occ0) - 15

        def zero_bins(_, __):
            def zb(i, _):
                bins[pl.ds(i * 16, 16)] = zero16
                return 0
            lax.fori_loop(0, 16, zb, 0)

        def publish_and_fetch_hist():
            pltpu.sync_copy(bins, hist_sh.at[pl.ds(tid * 256, 256)])
            plsc.subcore_barrier()
            pltpu.sync_copy(hist_sh, allhist)

        sc2.__exit__(None, None, None)
        sc3 = jax.named_scope("p3_select"); sc3.__enter__()
        P_hi = jnp.int32(0)
        K_rem = jnp.int32(K)
        n_gt = jnp.int32(0)
        for r, shift in enumerate(SHIFTS):
            zero_bins(0, 0)
            if r > 0:
                phh = lax.shift_right_logical(P_hi, shift + 8)

            def hist_body(j, _, _shift=shift, _r=r,
                          _phh=(None if r == 0 else phh)):
                for u in range(4):
                    ub = keys[pl.ds(j * 64 + u * 16, 16)] ^ INT_MIN
                    d = lax.shift_right_logical(ub, _shift) & 255
                    if _r == 0:
                        m = None
                    else:
                        m = lax.shift_right_logical(ub, _shift + 8) == _phh
                    occ, last = plsc.scan_count(d, mask=m)
                    cnt = occ - occ_base + 1
                    plsc.addupdate_scatter(bins, [d], cnt, mask=last)
                return 0

            sha = jax.named_scope(f"q_hist"); sha.__enter__()
            lax.fori_loop(0, CV // 4, hist_body, 0)
            sha.__exit__(None, None, None)
            shb = jax.named_scope(f"q_pub"); shb.__enter__()
            pltpu.sync_copy(bins, hist_sh.at[pl.ds(tid * 256, 256)])
            shb.__exit__(None, None, None)
            shc = jax.named_scope(f"q_bar"); shc.__enter__()
            plsc.subcore_barrier()
            shc.__exit__(None, None, None)
            shd = jax.named_scope(f"q_fetch"); shd.__enter__()
            pltpu.sync_copy(hist_sh, allhist)
            shd.__exit__(None, None, None)

            def comb_body(j, _):
                tot = zero16
                for t in range(NT):
                    tot = tot + allhist[pl.ds(t * 256 + j * 16, 16)]
                bins[pl.ds(j * 16, 16)] = tot
                return 0

            lax.fori_loop(0, 16, comb_body, 0)

            def suff_body(jj, carry):
                run, dstar = carry
                j = 15 - jj
                v = bins[pl.ds(j * 16, 16)]
                rv = lax.rev(v, (0,))
                cs = plsc.cumsum(rv) + run
                suff_incl = lax.rev(cs, (0,))
                dd = jnp.where(suff_incl >= K_rem, j * 16 + iota, -1)
                return jnp.max(cs), jnp.maximum(dstar, jnp.max(dd))

            _, dstar = lax.fori_loop(
                0, 16, suff_body, (jnp.int32(0), jnp.int32(-1)))

            def cntgt_body(j, acc):
                v = bins[pl.ds(j * 16, 16)]
                return acc + jnp.sum(jnp.where(j * 16 + iota > dstar, v, 0))

            cnt_gt = lax.fori_loop(0, 16, cntgt_body, jnp.int32(0))
            P_hi = P_hi | (dstar << shift)
            K_rem = K_rem - cnt_gt
            n_gt = n_gt + cnt_gt

        T = P_hi ^ INT_MIN
        K_eq = K_rem

        sc3.__exit__(None, None, None)
        sc4 = jax.named_scope("p4_compact"); sc4.__enter__()
        def comp_body(j, carry):
            gtc, eqc = carry
            u = keys[pl.ds(j * 16, 16)]
            rows16 = tid * C + j * 16 + iota
            mgt = u > T
            meq = u == T
            cgt = plsc.cumsum(jnp.where(mgt, 1, 0))
            ceq = plsc.cumsum(jnp.where(meq, 1, 0))
            dgt = gtc + cgt - 1
            deq = eqc + ceq - 1
            mgt_ok = mgt & (dgt < CAP)
            meq_ok = meq & (deq < CAP)
            plsc.store_scatter(gtkey, [dgt], u, mask=mgt_ok)
            plsc.store_scatter(gtidx, [dgt], rows16, mask=mgt_ok)
            plsc.store_scatter(eqidx, [deq], rows16, mask=meq_ok)
            return gtc + jnp.max(cgt), eqc + jnp.max(ceq)

        gtc, eqc = lax.fori_loop(
            0, CV, comp_body, (jnp.int32(0), jnp.int32(0)))
        gtc = jnp.minimum(gtc, CAP)
        eqc = jnp.minimum(eqc, CAP)

        cntv[...] = jnp.where(iota == 0, gtc, jnp.where(iota == 1, eqc, 0))
        pltpu.sync_copy(cntv, cnt_sh.at[pl.ds(tid * 16, 16)])
        plsc.subcore_barrier()
        pltpu.sync_copy(cnt_sh, allcnt)

        ngt_all = plsc.load_gather(allcnt, [iota * 16])
        neq_all = plsc.load_gather(allcnt, [iota * 16 + 1])
        csg = plsc.cumsum(ngt_all)
        cse = plsc.cumsum(neq_all)
        my_gt_off = jnp.sum(jnp.where(iota == tid, csg, 0)) - gtc
        my_eq_off = jnp.sum(jnp.where(iota == tid, cse, 0)) - eqc

        def eqd_body(i, _):
            s = i * 16 + iota
            rank = my_eq_off + s
            valid = (s < eqc) & (rank < K_eq)
            eqdst[pl.ds(i * 16, 16)] = jnp.where(valid, n_gt + rank, K + tid)
            return 0

        lax.fori_loop(0, CAPV, eqd_body, 0)
        pltpu.async_copy(eqidx, perm_sh.at[eqdst], sem).wait()

        sc4.__exit__(None, None, None)
        sc5 = jax.named_scope("p5_sort"); sc5.__enter__()
        for p, shift in enumerate((0, 8, 16, 24)):
            if p == 0:
                srckey, srcidx, srcdst, nv = gtkey, gtidx, gtdst, CAPV
            else:
                srckey, srcidx, srcdst, nv = entk, enti, entd, KV // 16
            dstkey_sh, dstidx_sh = (
                (akey_sh, aidx_sh) if p % 2 == 0 else (bkey_sh, bidx_sh))

            zero_bins(0, 0)

            def valid_mask(j, _p=p):
                s = j * 16 + iota
                if _p == 0:
                    return s < gtc
                return (tid * KV + s) < n_gt

            def hist_body2(j, _, _shift=shift, _src=srckey, _p=p):
                ub = _src[pl.ds(j * 16, 16)] ^ INT_MIN
                d = lax.shift_right_logical(ub, _shift) & 255
                m = valid_mask(j, _p)
                occ, last = plsc.scan_count(d, mask=m)
                plsc.addupdate_scatter(
                    bins, [d], occ - occ_base + 1, mask=last)
                return 0

            lax.fori_loop(0, nv, hist_body2, 0)
            publish_and_fetch_hist()

            def comb2_body(j, _):
                tot = zero16
                pre = zero16
                for t in range(NT):
                    av = allhist[pl.ds(t * 256 + j * 16, 16)]
                    tot = tot + av
                    pre = pre + jnp.where(t < tid, av, 0)
                bins[pl.ds(j * 16, 16)] = tot
                pret[pl.ds(j * 16, 16)] = pre
                return 0

            lax.fori_loop(0, 16, comb2_body, 0)

            def suff2_body(jj, run):
                j = 15 - jj
                v = bins[pl.ds(j * 16, 16)]
                rv = lax.rev(v, (0,))
                cs = plsc.cumsum(rv) + run
                suff_excl = lax.rev(cs, (0,)) - v
                offt[pl.ds(j * 16, 16)] = suff_excl + pret[pl.ds(j * 16, 16)]
                return jnp.max(cs)

            lax.fori_loop(0, 16, suff2_body, jnp.int32(0))

            def dst_body(j, _, _shift=shift, _srck=srckey, _srcd=srcdst,
                         _p=p):
                ub = _srck[pl.ds(j * 16, 16)] ^ INT_MIN
                d = lax.shift_right_logical(ub, _shift) & 255
                m = valid_mask(j, _p)
                base = plsc.load_gather(offt, [d])
                occ, last = plsc.scan_count(d, mask=m)
                occv = occ - occ_base
                _srcd[pl.ds(j * 16, 16)] = jnp.where(
                    m, base + occv, K + tid)
                plsc.store_scatter(offt, [d], base + occv + 1, mask=last)
                return 0

            lax.fori_loop(0, nv, dst_body, 0)

            if p < 3:
                pltpu.async_copy(srckey, dstkey_sh.at[srcdst], sem).wait()
                pltpu.async_copy(srcidx, dstidx_sh.at[srcdst], sem).wait()
                plsc.subcore_barrier()
                pltpu.sync_copy(dstkey_sh.at[pl.ds(tid * KV, KV)], entk)
                pltpu.sync_copy(dstidx_sh.at[pl.ds(tid * KV, KV)], enti)
            else:
                pltpu.async_copy(srcidx, perm_sh.at[srcdst], sem).wait()
                plsc.subcore_barrier()

        sc5.__exit__(None, None, None)
        sc6 = jax.named_scope("p6_gather"); sc6.__enter__()
        pltpu.sync_copy(perm_sh.at[pl.ds(tid * KV, KV)], myperm)
        row_copies = []
        for g in range(KV // 16):
            pv = myperm[pl.ds(g * 16, 16)]
            for l in range(16):
                i = g * 16 + l
                r = pv[l]
                row_copies.append(pltpu.async_copy(
                    xf_hbm.at[pl.ds(r * D, D)], rows.at[i], sem))
        for cp in row_copies:
            cp.wait()
        pltpu.sync_copy(rows, out_hbm.at[pl.ds(tid * KV, KV)])
        sc6.__exit__(None, None, None)

    return k(Xf)


def kernel(X):
    return _topk_perm(X.reshape(-1))


# speedup vs baseline: 1.0977x; 1.0977x over previous
import functools

import jax
import jax.numpy as jnp
from jax import lax
from jax.experimental import pallas as pl
from jax.experimental.pallas import tpu as pltpu
from jax.experimental.pallas import tpu_sc as plsc

N, D, K = 32768, 128, 1024
NT = 16
C = N // NT
CV = C // 16
CAP = 256
CAPV = CAP // 16
KV = K // NT
SHIFTS = (24, 16, 8, 0)
INT_MIN = -(2**31)


def _topk_perm(Xf):
    mesh = plsc.VectorSubcoreMesh(
        core_axis_name="c", subcore_axis_name="s", num_cores=1)

    @functools.partial(
        pl.kernel,
        out_type=jax.ShapeDtypeStruct((K, D), jnp.float32),
        mesh=mesh,
        compiler_params=pltpu.CompilerParams(needs_layout_passes=False),
        scratch_types=[
            pltpu.VMEM((C,), jnp.float32),
            pltpu.VMEM((C,), jnp.int32),
            pltpu.VMEM((C,), jnp.int32),
            pltpu.VMEM((256,), jnp.int32),
            pltpu.VMEM((256,), jnp.int32),
            pltpu.VMEM((256,), jnp.int32),
            pltpu.VMEM((NT * 256,), jnp.int32),
            pltpu.VMEM((16,), jnp.int32),
            pltpu.VMEM((NT * 16,), jnp.int32),
            pltpu.VMEM((CAP,), jnp.int32),
            pltpu.VMEM((CAP,), jnp.int32),
            pltpu.VMEM((CAP,), jnp.int32),
            pltpu.VMEM((CAP,), jnp.int32),
            pltpu.VMEM((CAP,), jnp.int32),
            pltpu.VMEM((KV,), jnp.int32),
            pltpu.VMEM((KV,), jnp.int32),
            pltpu.VMEM((KV,), jnp.int32),
            pltpu.VMEM((KV,), jnp.int32),
            pltpu.VMEM((KV, D), jnp.float32),
            pltpu.VMEM_SHARED((NT * 256,), jnp.int32),
            pltpu.VMEM_SHARED((NT * 16,), jnp.int32),
            pltpu.VMEM_SHARED((K + NT,), jnp.int32),
            pltpu.VMEM_SHARED((K + NT,), jnp.int32),
            pltpu.VMEM_SHARED((K + NT,), jnp.int32),
            pltpu.VMEM_SHARED((K + NT,), jnp.int32),
            pltpu.VMEM_SHARED((K + NT,), jnp.int32),
            pltpu.SemaphoreType.DMA,
        ],
    )
    def k(xf_hbm, out_hbm, colflat, idxcol, keys, bins, pret,
          offt, allhist, cntv,
          allcnt, gtkey, gtidx, gtdst, eqidx, eqdst, entk, enti, entd,
          myperm, rows, hist_sh, cnt_sh, akey_sh, aidx_sh, bkey_sh,
          bidx_sh, perm_sh, sem):
        tid = lax.axis_index("s")
        iota = lax.iota(jnp.int32, 16)
        zero16 = iota ^ iota
        one16 = zero16 + 1

        sc1 = jax.named_scope("p1_col"); sc1.__enter__()
        def idx_body(j, _):
            for u in range(4):
                base = j * 64 + u * 16
                idxcol[pl.ds(base, 16)] = (tid * C + base + iota) * D + (
                    D - 1)
            return 0

        lax.fori_loop(0, CV // 4, idx_body, 0)
        col_copies = [
            pltpu.async_copy(
                xf_hbm.at[idxcol.at[pl.ds(i * 128, 128)]],
                colflat.at[pl.ds(i * 128, 128)], sem)
            for i in range(16)
        ]
        for cp in col_copies:
            cp.wait()

        sc1.__exit__(None, None, None)
        sc2 = jax.named_scope("p2_keys"); sc2.__enter__()
        def keys_body(j, _):
            for u in range(4):
                base = j * 64 + u * 16
                v = colflat[pl.ds(base, 16)]
                b = plsc.bitcast(v, jnp.int32)
                big = jnp.where(b >= 0, b, ~b ^ INT_MIN)
                big = jnp.where(b == INT_MIN, 0, big)
                keys[pl.ds(base, 16)] = big
            return 0

        lax.fori_loop(0, CV // 4, keys_body, 0)

        occ0, _ = plsc.scan_count(zero16)
        occ_base = jnp.max(occ0) - 15

        def zero_bins(_, __):
            def zb(i, _):
                bins[pl.ds(i * 16, 16)] = zero16
                return 0
            lax.fori_loop(0, 16, zb, 0)

        def publish_and_fetch_hist():
            pltpu.sync_copy(bins, hist_sh.at[pl.ds(tid * 256, 256)])
            plsc.subcore_barrier()
            pltpu.sync_copy(hist_sh, allhist)

        sc2.__exit__(None, None, None)
        sc3 = jax.named_scope("p3_select"); sc3.__enter__()
        P_hi = jnp.int32(0)
        K_rem = jnp.int32(K)
        n_gt = jnp.int32(0)
        for r, shift in enumerate(SHIFTS):
            zero_bins(0, 0)
            if r > 0:
                phh = lax.shift_right_logical(P_hi, shift + 8)

            def hist_body(j, _, _shift=shift, _r=r,
                          _phh=(None if r == 0 else phh)):
                for u in range(4):
                    ub = keys[pl.ds(j * 64 + u * 16, 16)] ^ INT_MIN
                    d = lax.shift_right_logical(ub, _shift) & 255
                    if _r == 0:
                        m = None
                    else:
                        m = lax.shift_right_logical(ub, _shift + 8) == _phh
                    plsc.addupdate_scatter(bins, [d], one16, mask=m)
                return 0

            sha = jax.named_scope(f"q_hist"); sha.__enter__()
            lax.fori_loop(0, CV // 4, hist_body, 0)
            sha.__exit__(None, None, None)
            shb = jax.named_scope(f"q_pub"); shb.__enter__()
            pltpu.sync_copy(bins, hist_sh.at[pl.ds(tid * 256, 256)])
            shb.__exit__(None, None, None)
            shc = jax.named_scope(f"q_bar"); shc.__enter__()
            plsc.subcore_barrier()
            shc.__exit__(None, None, None)
            shd = jax.named_scope(f"q_fetch"); shd.__enter__()
            pltpu.sync_copy(hist_sh, allhist)
            shd.__exit__(None, None, None)

            def comb_body(j, _):
                tot = zero16
                for t in range(NT):
                    tot = tot + allhist[pl.ds(t * 256 + j * 16, 16)]
                bins[pl.ds(j * 16, 16)] = tot
                return 0

            lax.fori_loop(0, 16, comb_body, 0)

            def suff_body(jj, carry):
                run, dstar = carry
                j = 15 - jj
                v = bins[pl.ds(j * 16, 16)]
                rv = lax.rev(v, (0,))
                cs = plsc.cumsum(rv) + run
                suff_incl = lax.rev(cs, (0,))
                dd = jnp.where(suff_incl >= K_rem, j * 16 + iota, -1)
                return jnp.max(cs), jnp.maximum(dstar, jnp.max(dd))

            _, dstar = lax.fori_loop(
                0, 16, suff_body, (jnp.int32(0), jnp.int32(-1)))

            def cntgt_body(j, acc):
                v = bins[pl.ds(j * 16, 16)]
                return acc + jnp.sum(jnp.where(j * 16 + iota > dstar, v, 0))

            cnt_gt = lax.fori_loop(0, 16, cntgt_body, jnp.int32(0))
            P_hi = P_hi | (dstar << shift)
            K_rem = K_rem - cnt_gt
            n_gt = n_gt + cnt_gt

        T = P_hi ^ INT_MIN
        K_eq = K_rem

        sc3.__exit__(None, None, None)
        sc4 = jax.named_scope("p4_compact"); sc4.__enter__()
        def comp_body(j, carry):
            gtc, eqc = carry
            u = keys[pl.ds(j * 16, 16)]
            rows16 = tid * C + j * 16 + iota
            mgt = u > T
            meq = u == T
            cgt = plsc.cumsum(jnp.where(mgt, 1, 0))
            ceq = plsc.cumsum(jnp.where(meq, 1, 0))
            dgt = gtc + cgt - 1
            deq = eqc + ceq - 1
            mgt_ok = mgt & (dgt < CAP)
            meq_ok = meq & (deq < CAP)
            plsc.store_scatter(gtkey, [dgt], u, mask=mgt_ok)
            plsc.store_scatter(gtidx, [dgt], rows16, mask=mgt_ok)
            plsc.store_scatter(eqidx, [deq], rows16, mask=meq_ok)
            return gtc + jnp.max(cgt), eqc + jnp.max(ceq)

        gtc, eqc = lax.fori_loop(
            0, CV, comp_body, (jnp.int32(0), jnp.int32(0)))
        gtc = jnp.minimum(gtc, CAP)
        eqc = jnp.minimum(eqc, CAP)

        cntv[...] = jnp.where(iota == 0, gtc, jnp.where(iota == 1, eqc, 0))
        pltpu.sync_copy(cntv, cnt_sh.at[pl.ds(tid * 16, 16)])
        plsc.subcore_barrier()
        pltpu.sync_copy(cnt_sh, allcnt)

        ngt_all = plsc.load_gather(allcnt, [iota * 16])
        neq_all = plsc.load_gather(allcnt, [iota * 16 + 1])
        csg = plsc.cumsum(ngt_all)
        cse = plsc.cumsum(neq_all)
        my_gt_off = jnp.sum(jnp.where(iota == tid, csg, 0)) - gtc
        my_eq_off = jnp.sum(jnp.where(iota == tid, cse, 0)) - eqc

        def eqd_body(i, _):
            s = i * 16 + iota
            rank = my_eq_off + s
            valid = (s < eqc) & (rank < K_eq)
            eqdst[pl.ds(i * 16, 16)] = jnp.where(valid, n_gt + rank, K + tid)
            return 0

        lax.fori_loop(0, CAPV, eqd_body, 0)
        pltpu.async_copy(eqidx, perm_sh.at[eqdst], sem).wait()

        sc4.__exit__(None, None, None)
        sc5 = jax.named_scope("p5_sort"); sc5.__enter__()
        for p, shift in enumerate((0, 8, 16, 24)):
            if p == 0:
                srckey, srcidx, srcdst, nv = gtkey, gtidx, gtdst, CAPV
            else:
                srckey, srcidx, srcdst, nv = entk, enti, entd, KV // 16
            dstkey_sh, dstidx_sh = (
                (akey_sh, aidx_sh) if p % 2 == 0 else (bkey_sh, bidx_sh))

            zero_bins(0, 0)

            def valid_mask(j, _p=p):
                s = j * 16 + iota
                if _p == 0:
                    return s < gtc
                return (tid * KV + s) < n_gt

            def hist_body2(j, _, _shift=shift, _src=srckey, _p=p):
                ub = _src[pl.ds(j * 16, 16)] ^ INT_MIN
                d = lax.shift_right_logical(ub, _shift) & 255
                m = valid_mask(j, _p)
                plsc.addupdate_scatter(bins, [d], one16, mask=m)
                return 0

            lax.fori_loop(0, nv, hist_body2, 0)
            publish_and_fetch_hist()

            def comb2_body(j, _):
                tot = zero16
                pre = zero16
                for t in range(NT):
                    av = allhist[pl.ds(t * 256 + j * 16, 16)]
                    tot = tot + av
                    pre = pre + jnp.where(t < tid, av, 0)
                bins[pl.ds(j * 16, 16)] = tot
                pret[pl.ds(j * 16, 16)] = pre
                return 0

            lax.fori_loop(0, 16, comb2_body, 0)

            def suff2_body(jj, run):
                j = 15 - jj
                v = bins[pl.ds(j * 16, 16)]
                rv = lax.rev(v, (0,))
                cs = plsc.cumsum(rv) + run
                suff_excl = lax.rev(cs, (0,)) - v
                offt[pl.ds(j * 16, 16)] = suff_excl + pret[pl.ds(j * 16, 16)]
                return jnp.max(cs)

            lax.fori_loop(0, 16, suff2_body, jnp.int32(0))

            def dst_body(j, _, _shift=shift, _srck=srckey, _srcd=srcdst,
                         _p=p):
                ub = _srck[pl.ds(j * 16, 16)] ^ INT_MIN
                d = lax.shift_right_logical(ub, _shift) & 255
                m = valid_mask(j, _p)
                base = plsc.load_gather(offt, [d])
                occ, last = plsc.scan_count(d, mask=m)
                occv = occ - occ_base
                _srcd[pl.ds(j * 16, 16)] = jnp.where(
                    m, base + occv, K + tid)
                plsc.store_scatter(offt, [d], base + occv + 1, mask=last)
                return 0

            lax.fori_loop(0, nv, dst_body, 0)

            if p < 3:
                pltpu.async_copy(srckey, dstkey_sh.at[srcdst], sem).wait()
                pltpu.async_copy(srcidx, dstidx_sh.at[srcdst], sem).wait()
                plsc.subcore_barrier()
                pltpu.sync_copy(dstkey_sh.at[pl.ds(tid * KV, KV)], entk)
                pltpu.sync_copy(dstidx_sh.at[pl.ds(tid * KV, KV)], enti)
            else:
                pltpu.async_copy(srcidx, perm_sh.at[srcdst], sem).wait()
                plsc.subcore_barrier()

        sc5.__exit__(None, None, None)
        sc6 = jax.named_scope("p6_gather"); sc6.__enter__()
        pltpu.sync_copy(perm_sh.at[pl.ds(tid * KV, KV)], myperm)
        row_copies = []
        for g in range(KV // 16):
            pv = myperm[pl.ds(g * 16, 16)]
            for l in range(16):
                i = g * 16 + l
                r = pv[l]
                row_copies.append(pltpu.async_copy(
                    xf_hbm.at[pl.ds(r * D, D)], rows.at[i], sem))
        for cp in row_copies:
            cp.wait()
        pltpu.sync_copy(rows, out_hbm.at[pl.ds(tid * KV, KV)])
        sc6.__exit__(None, None, None)

    return k(Xf)


def kernel(X):
    return _topk_perm(X.reshape(-1))


# speedup vs baseline: 1.1027x; 1.0046x over previous
import functools

import jax
import jax.numpy as jnp
from jax import lax
from jax.experimental import pallas as pl
from jax.experimental.pallas import tpu as pltpu
from jax.experimental.pallas import tpu_sc as plsc

N, D, K = 32768, 128, 1024
NT = 16
C = N // NT
CV = C // 16
CAP = 256
CAPV = CAP // 16
KV = K // NT
SHIFTS = (24, 16, 8, 0)
INT_MIN = -(2**31)


def _topk_perm(Xf):
    mesh = plsc.VectorSubcoreMesh(
        core_axis_name="c", subcore_axis_name="s", num_cores=1)

    @functools.partial(
        pl.kernel,
        out_type=jax.ShapeDtypeStruct((K, D), jnp.float32),
        mesh=mesh,
        compiler_params=pltpu.CompilerParams(needs_layout_passes=False),
        scratch_types=[
            pltpu.VMEM((C,), jnp.float32),
            pltpu.VMEM((C,), jnp.int32),
            pltpu.VMEM((C,), jnp.int32),
            pltpu.VMEM((256,), jnp.int32),
            pltpu.VMEM((256,), jnp.int32),
            pltpu.VMEM((256,), jnp.int32),
            pltpu.VMEM((NT * 256,), jnp.int32),
            pltpu.VMEM((16,), jnp.int32),
            pltpu.VMEM((NT * 16,), jnp.int32),
            pltpu.VMEM((CAP,), jnp.int32),
            pltpu.VMEM((CAP,), jnp.int32),
            pltpu.VMEM((CAP,), jnp.int32),
            pltpu.VMEM((CAP,), jnp.int32),
            pltpu.VMEM((CAP,), jnp.int32),
            pltpu.VMEM((KV,), jnp.int32),
            pltpu.VMEM((KV,), jnp.int32),
            pltpu.VMEM((KV,), jnp.int32),
            pltpu.VMEM((KV,), jnp.int32),
            pltpu.VMEM((KV, D), jnp.float32),
            pltpu.VMEM_SHARED((NT * 256,), jnp.int32),
            pltpu.VMEM_SHARED((NT * 16,), jnp.int32),
            pltpu.VMEM_SHARED((K + NT,), jnp.int32),
            pltpu.VMEM_SHARED((K + NT,), jnp.int32),
            pltpu.VMEM_SHARED((K + NT,), jnp.int32),
            pltpu.VMEM_SHARED((K + NT,), jnp.int32),
            pltpu.VMEM_SHARED((K + NT,), jnp.int32),
            pltpu.SemaphoreType.DMA,
        ],
    )
    def k(xf_hbm, out_hbm, colflat, idxcol, keys, bins, pret,
          offt, allhist, cntv,
          allcnt, gtkey, gtidx, gtdst, eqidx, eqdst, entk, enti, entd,
          myperm, rows, hist_sh, cnt_sh, akey_sh, aidx_sh, bkey_sh,
          bidx_sh, perm_sh, sem):
        tid = lax.axis_index("s")
        iota = lax.iota(jnp.int32, 16)
        zero16 = iota ^ iota
        one16 = zero16 + 1

        def idx_body(j, _):
            for u in range(4):
                base = j * 64 + u * 16
                idxcol[pl.ds(base, 16)] = (tid * C + base + iota) * D + (
                    D - 1)
            return 0

        lax.fori_loop(0, CV // 4, idx_body, 0)
        col_copies = [
            pltpu.async_copy(
                xf_hbm.at[idxcol.at[pl.ds(i * 128, 128)]],
                colflat.at[pl.ds(i * 128, 128)], sem)
            for i in range(16)
        ]
        for cp in col_copies:
            cp.wait()

        def keys_body(j, _):
            for u in range(4):
                base = j * 64 + u * 16
                v = colflat[pl.ds(base, 16)]
                b = plsc.bitcast(v, jnp.int32)
                big = jnp.where(b >= 0, b, ~b ^ INT_MIN)
                big = jnp.where(b == INT_MIN, 0, big)
                keys[pl.ds(base, 16)] = big
            return 0

        lax.fori_loop(0, CV // 4, keys_body, 0)

        occ0, _ = plsc.scan_count(zero16)
        occ_base = jnp.max(occ0) - 15

        def zero_bins(_, __):
            def zb(i, _):
                bins[pl.ds(i * 16, 16)] = zero16
                return 0
            lax.fori_loop(0, 16, zb, 0)

        def publish_and_fetch_hist():
            pltpu.sync_copy(bins, hist_sh.at[pl.ds(tid * 256, 256)])
            plsc.subcore_barrier()
            pltpu.sync_copy(hist_sh, allhist)

        P_hi = jnp.int32(0)
        K_rem = jnp.int32(K)
        n_gt = jnp.int32(0)
        for r, shift in enumerate(SHIFTS):
            zero_bins(0, 0)
            if r > 0:
                phh = lax.shift_right_logical(P_hi, shift + 8)

            def hist_body(j, _, _shift=shift, _r=r,
                          _phh=(None if r == 0 else phh)):
                for u in range(4):
                    ub = keys[pl.ds(j * 64 + u * 16, 16)] ^ INT_MIN
                    d = lax.shift_right_logical(ub, _shift) & 255
                    if _r == 0:
                        m = None
                    else:
                        m = lax.shift_right_logical(ub, _shift + 8) == _phh
                    plsc.addupdate_scatter(bins, [d], one16, mask=m)
                return 0

            lax.fori_loop(0, CV // 4, hist_body, 0)
            publish_and_fetch_hist()

            def comb_body(j, _):
                tot = zero16
                for t in range(NT):
                    tot = tot + allhist[pl.ds(t * 256 + j * 16, 16)]
                bins[pl.ds(j * 16, 16)] = tot
                return 0

            lax.fori_loop(0, 16, comb_body, 0)

            def suff_body(jj, carry):
                run, dstar = carry
                j = 15 - jj
                v = bins[pl.ds(j * 16, 16)]
                rv = lax.rev(v, (0,))
                cs = plsc.cumsum(rv) + run
                suff_incl = lax.rev(cs, (0,))
                dd = jnp.where(suff_incl >= K_rem, j * 16 + iota, -1)
                return jnp.max(cs), jnp.maximum(dstar, jnp.max(dd))

            _, dstar = lax.fori_loop(
                0, 16, suff_body, (jnp.int32(0), jnp.int32(-1)))

            def cntgt_body(j, acc):
                v = bins[pl.ds(j * 16, 16)]
                return acc + jnp.sum(jnp.where(j * 16 + iota > dstar, v, 0))

            cnt_gt = lax.fori_loop(0, 16, cntgt_body, jnp.int32(0))
            P_hi = P_hi | (dstar << shift)
            K_rem = K_rem - cnt_gt
            n_gt = n_gt + cnt_gt

        T = P_hi ^ INT_MIN
        K_eq = K_rem

        def comp_body(j, carry):
            gtc, eqc = carry
            u = keys[pl.ds(j * 16, 16)]
            rows16 = tid * C + j * 16 + iota
            mgt = u > T
            meq = u == T
            cgt = plsc.cumsum(jnp.where(mgt, 1, 0))
            ceq = plsc.cumsum(jnp.where(meq, 1, 0))
            dgt = gtc + cgt - 1
            deq = eqc + ceq - 1
            mgt_ok = mgt & (dgt < CAP)
            meq_ok = meq & (deq < CAP)
            plsc.store_scatter(gtkey, [dgt], u, mask=mgt_ok)
            plsc.store_scatter(gtidx, [dgt], rows16, mask=mgt_ok)
            plsc.store_scatter(eqidx, [deq], rows16, mask=meq_ok)
            return gtc + jnp.max(cgt), eqc + jnp.max(ceq)

        gtc, eqc = lax.fori_loop(
            0, CV, comp_body, (jnp.int32(0), jnp.int32(0)))
        gtc = jnp.minimum(gtc, CAP)
        eqc = jnp.minimum(eqc, CAP)

        cntv[...] = jnp.where(iota == 0, gtc, jnp.where(iota == 1, eqc, 0))
        pltpu.sync_copy(cntv, cnt_sh.at[pl.ds(tid * 16, 16)])
        plsc.subcore_barrier()
        pltpu.sync_copy(cnt_sh, allcnt)

        ngt_all = plsc.load_gather(allcnt, [iota * 16])
        neq_all = plsc.load_gather(allcnt, [iota * 16 + 1])
        csg = plsc.cumsum(ngt_all)
        cse = plsc.cumsum(neq_all)
        my_gt_off = jnp.sum(jnp.where(iota == tid, csg, 0)) - gtc
        my_eq_off = jnp.sum(jnp.where(iota == tid, cse, 0)) - eqc

        def eqd_body(i, _):
            s = i * 16 + iota
            rank = my_eq_off + s
            valid = (s < eqc) & (rank < K_eq)
            eqdst[pl.ds(i * 16, 16)] = jnp.where(valid, n_gt + rank, K + tid)
            return 0

        lax.fori_loop(0, CAPV, eqd_body, 0)
        pltpu.async_copy(eqidx, perm_sh.at[eqdst], sem).wait()

        for p, shift in enumerate((0, 8, 16, 24)):
            if p == 0:
                srckey, srcidx, srcdst, nv = gtkey, gtidx, gtdst, CAPV
            else:
                srckey, srcidx, srcdst, nv = entk, enti, entd, KV // 16
            dstkey_sh, dstidx_sh = (
                (akey_sh, aidx_sh) if p % 2 == 0 else (bkey_sh, bidx_sh))

            zero_bins(0, 0)

            def valid_mask(j, _p=p):
                s = j * 16 + iota
                if _p == 0:
                    return s < gtc
                return (tid * KV + s) < n_gt

            def hist_body2(j, _, _shift=shift, _src=srckey, _p=p):
                ub = _src[pl.ds(j * 16, 16)] ^ INT_MIN
                d = lax.shift_right_logical(ub, _shift) & 255
                m = valid_mask(j, _p)
                plsc.addupdate_scatter(bins, [d], one16, mask=m)
                return 0

            lax.fori_loop(0, nv, hist_body2, 0)
            publish_and_fetch_hist()

            def comb2_body(j, _):
                tot = zero16
                pre = zero16
                for t in range(NT):
                    av = allhist[pl.ds(t * 256 + j * 16, 16)]
                    tot = tot + av
                    pre = pre + jnp.where(t < tid, av, 0)
                bins[pl.ds(j * 16, 16)] = tot
                pret[pl.ds(j * 16, 16)] = pre
                return 0

            lax.fori_loop(0, 16, comb2_body, 0)

            def suff2_body(jj, run):
                j = 15 - jj
                v = bins[pl.ds(j * 16, 16)]
                rv = lax.rev(v, (0,))
                cs = plsc.cumsum(rv) + run
                suff_excl = lax.rev(cs, (0,)) - v
                offt[pl.ds(j * 16, 16)] = suff_excl + pret[pl.ds(j * 16, 16)]
                return jnp.max(cs)

            lax.fori_loop(0, 16, suff2_body, jnp.int32(0))

            def dst_body(j, _, _shift=shift, _srck=srckey, _srcd=srcdst,
                         _p=p):
                ub = _srck[pl.ds(j * 16, 16)] ^ INT_MIN
                d = lax.shift_right_logical(ub, _shift) & 255
                m = valid_mask(j, _p)
                base = plsc.load_gather(offt, [d])
                occ, last = plsc.scan_count(d, mask=m)
                occv = occ - occ_base
                _srcd[pl.ds(j * 16, 16)] = jnp.where(
                    m, base + occv, K + tid)
                plsc.store_scatter(offt, [d], base + occv + 1, mask=last)
                return 0

            lax.fori_loop(0, nv, dst_body, 0)

            if p < 3:
                pltpu.async_copy(srckey, dstkey_sh.at[srcdst], sem).wait()
                pltpu.async_copy(srcidx, dstidx_sh.at[srcdst], sem).wait()
                plsc.subcore_barrier()
                pltpu.sync_copy(dstkey_sh.at[pl.ds(tid * KV, KV)], entk)
                pltpu.sync_copy(dstidx_sh.at[pl.ds(tid * KV, KV)], enti)
            else:
                pltpu.async_copy(srcidx, perm_sh.at[srcdst], sem).wait()
                plsc.subcore_barrier()

        pltpu.sync_copy(perm_sh.at[pl.ds(tid * KV, KV)], myperm)
        row_copies = []
        for g in range(KV // 16):
            pv = myperm[pl.ds(g * 16, 16)]
            for l in range(16):
                i = g * 16 + l
                r = pv[l]
                row_copies.append(pltpu.async_copy(
                    xf_hbm.at[pl.ds(r * D, D)], rows.at[i], sem))
        for cp in row_copies:
            cp.wait()
        pltpu.sync_copy(rows, out_hbm.at[pl.ds(tid * KV, KV)])

    return k(Xf)


def kernel(X):
    return _topk_perm(X.reshape(-1))


# speedup vs baseline: 1.1029x; 1.0002x over previous
import functools

import jax
import jax.numpy as jnp
from jax import lax
from jax.experimental import pallas as pl
from jax.experimental.pallas import tpu as pltpu
from jax.experimental.pallas import tpu_sc as plsc

N, D, K = 32768, 128, 1024
NT = 16
C = N // NT
CV = C // 16
CAP = 256
CAPV = CAP // 16
KV = K // NT
SHIFTS = (24, 16, 8, 0)
INT_MIN = -(2**31)


def _topk_perm(Xf):
    mesh = plsc.VectorSubcoreMesh(
        core_axis_name="c", subcore_axis_name="s", num_cores=1)

    @functools.partial(
        pl.kernel,
        out_type=jax.ShapeDtypeStruct((K, D), jnp.float32),
        mesh=mesh,
        compiler_params=pltpu.CompilerParams(needs_layout_passes=False, disable_bounds_checks=True, disable_semaphore_checks=True),
        scratch_types=[
            pltpu.VMEM((C,), jnp.float32),
            pltpu.VMEM((C,), jnp.int32),
            pltpu.VMEM((C,), jnp.int32),
            pltpu.VMEM((256,), jnp.int32),
            pltpu.VMEM((256,), jnp.int32),
            pltpu.VMEM((256,), jnp.int32),
            pltpu.VMEM((NT * 256,), jnp.int32),
            pltpu.VMEM((16,), jnp.int32),
            pltpu.VMEM((NT * 16,), jnp.int32),
            pltpu.VMEM((CAP,), jnp.int32),
            pltpu.VMEM((CAP,), jnp.int32),
            pltpu.VMEM((CAP,), jnp.int32),
            pltpu.VMEM((CAP,), jnp.int32),
            pltpu.VMEM((CAP,), jnp.int32),
            pltpu.VMEM((KV,), jnp.int32),
            pltpu.VMEM((KV,), jnp.int32),
            pltpu.VMEM((KV,), jnp.int32),
            pltpu.VMEM((KV,), jnp.int32),
            pltpu.VMEM((KV, D), jnp.float32),
            pltpu.VMEM_SHARED((NT * 256,), jnp.int32),
            pltpu.VMEM_SHARED((NT * 16,), jnp.int32),
            pltpu.VMEM_SHARED((K + NT,), jnp.int32),
            pltpu.VMEM_SHARED((K + NT,), jnp.int32),
            pltpu.VMEM_SHARED((K + NT,), jnp.int32),
            pltpu.VMEM_SHARED((K + NT,), jnp.int32),
            pltpu.VMEM_SHARED((K + NT,), jnp.int32),
            pltpu.SemaphoreType.DMA,
        ],
    )
    def k(xf_hbm, out_hbm, colflat, idxcol, keys, bins, pret,
          offt, allhist, cntv,
          allcnt, gtkey, gtidx, gtdst, eqidx, eqdst, entk, enti, entd,
          myperm, rows, hist_sh, cnt_sh, akey_sh, aidx_sh, bkey_sh,
          bidx_sh, perm_sh, sem):
        tid = lax.axis_index("s")
        iota = lax.iota(jnp.int32, 16)
        zero16 = iota ^ iota
        one16 = zero16 + 1

        def idx_body(j, _):
            for u in range(4):
                base = j * 64 + u * 16
                idxcol[pl.ds(base, 16)] = (tid * C + base + iota) * D + (
                    D - 1)
            return 0

        lax.fori_loop(0, CV // 4, idx_body, 0)
        col_copies = [
            pltpu.async_copy(
                xf_hbm.at[idxcol.at[pl.ds(i * 128, 128)]],
                colflat.at[pl.ds(i * 128, 128)], sem)
            for i in range(16)
        ]
        for cp in col_copies:
            cp.wait()

        def keys_body(j, _):
            for u in range(4):
                base = j * 64 + u * 16
                v = colflat[pl.ds(base, 16)]
                b = plsc.bitcast(v, jnp.int32)
                big = jnp.where(b >= 0, b, ~b ^ INT_MIN)
                big = jnp.where(b == INT_MIN, 0, big)
                keys[pl.ds(base, 16)] = big
            return 0

        lax.fori_loop(0, CV // 4, keys_body, 0)

        occ0, _ = plsc.scan_count(zero16)
        occ_base = jnp.max(occ0) - 15

        def zero_bins(_, __):
            def zb(i, _):
                bins[pl.ds(i * 16, 16)] = zero16
                return 0
            lax.fori_loop(0, 16, zb, 0)

        def publish_and_fetch_hist():
            pltpu.sync_copy(bins, hist_sh.at[pl.ds(tid * 256, 256)])
            plsc.subcore_barrier()
            pltpu.sync_copy(hist_sh, allhist)

        P_hi = jnp.int32(0)
        K_rem = jnp.int32(K)
        n_gt = jnp.int32(0)
        for r, shift in enumerate(SHIFTS):
            zero_bins(0, 0)
            if r > 0:
                phh = lax.shift_right_logical(P_hi, shift + 8)

            def hist_body(j, _, _shift=shift, _r=r,
                          _phh=(None if r == 0 else phh)):
                for u in range(4):
                    ub = keys[pl.ds(j * 64 + u * 16, 16)] ^ INT_MIN
                    d = lax.shift_right_logical(ub, _shift) & 255
                    if _r == 0:
                        m = None
                    else:
                        m = lax.shift_right_logical(ub, _shift + 8) == _phh
                    plsc.addupdate_scatter(bins, [d], one16, mask=m)
                return 0

            lax.fori_loop(0, CV // 4, hist_body, 0)
            publish_and_fetch_hist()

            def comb_body(j, _):
                tot = zero16
                for t in range(NT):
                    tot = tot + allhist[pl.ds(t * 256 + j * 16, 16)]
                bins[pl.ds(j * 16, 16)] = tot
                return 0

            lax.fori_loop(0, 16, comb_body, 0)

            def suff_body(jj, carry):
                run, dstar = carry
                j = 15 - jj
                v = bins[pl.ds(j * 16, 16)]
                rv = lax.rev(v, (0,))
                cs = plsc.cumsum(rv) + run
                suff_incl = lax.rev(cs, (0,))
                dd = jnp.where(suff_incl >= K_rem, j * 16 + iota, -1)
                return jnp.max(cs), jnp.maximum(dstar, jnp.max(dd))

            _, dstar = lax.fori_loop(
                0, 16, suff_body, (jnp.int32(0), jnp.int32(-1)))

            def cntgt_body(j, acc):
                v = bins[pl.ds(j * 16, 16)]
                return acc + jnp.sum(jnp.where(j * 16 + iota > dstar, v, 0))

            cnt_gt = lax.fori_loop(0, 16, cntgt_body, jnp.int32(0))
            P_hi = P_hi | (dstar << shift)
            K_rem = K_rem - cnt_gt
            n_gt = n_gt + cnt_gt

        T = P_hi ^ INT_MIN
        K_eq = K_rem

        def comp_body(j, carry):
            gtc, eqc = carry
            u = keys[pl.ds(j * 16, 16)]
            rows16 = tid * C + j * 16 + iota
            mgt = u > T
            meq = u == T
            cgt = plsc.cumsum(jnp.where(mgt, 1, 0))
            ceq = plsc.cumsum(jnp.where(meq, 1, 0))
            dgt = gtc + cgt - 1
            deq = eqc + ceq - 1
            mgt_ok = mgt & (dgt < CAP)
            meq_ok = meq & (deq < CAP)
            plsc.store_scatter(gtkey, [dgt], u, mask=mgt_ok)
            plsc.store_scatter(gtidx, [dgt], rows16, mask=mgt_ok)
            plsc.store_scatter(eqidx, [deq], rows16, mask=meq_ok)
            return gtc + jnp.max(cgt), eqc + jnp.max(ceq)

        gtc, eqc = lax.fori_loop(
            0, CV, comp_body, (jnp.int32(0), jnp.int32(0)))
        gtc = jnp.minimum(gtc, CAP)
        eqc = jnp.minimum(eqc, CAP)

        cntv[...] = jnp.where(iota == 0, gtc, jnp.where(iota == 1, eqc, 0))
        pltpu.sync_copy(cntv, cnt_sh.at[pl.ds(tid * 16, 16)])
        plsc.subcore_barrier()
        pltpu.sync_copy(cnt_sh, allcnt)

        ngt_all = plsc.load_gather(allcnt, [iota * 16])
        neq_all = plsc.load_gather(allcnt, [iota * 16 + 1])
        csg = plsc.cumsum(ngt_all)
        cse = plsc.cumsum(neq_all)
        my_gt_off = jnp.sum(jnp.where(iota == tid, csg, 0)) - gtc
        my_eq_off = jnp.sum(jnp.where(iota == tid, cse, 0)) - eqc

        def eqd_body(i, _):
            s = i * 16 + iota
            rank = my_eq_off + s
            valid = (s < eqc) & (rank < K_eq)
            eqdst[pl.ds(i * 16, 16)] = jnp.where(valid, n_gt + rank, K + tid)
            return 0

        lax.fori_loop(0, CAPV, eqd_body, 0)
        pltpu.async_copy(eqidx, perm_sh.at[eqdst], sem).wait()

        for p, shift in enumerate((0, 8, 16, 24)):
            if p == 0:
                srckey, srcidx, srcdst, nv = gtkey, gtidx, gtdst, CAPV
            else:
                srckey, srcidx, srcdst, nv = entk, enti, entd, KV // 16
            dstkey_sh, dstidx_sh = (
                (akey_sh, aidx_sh) if p % 2 == 0 else (bkey_sh, bidx_sh))

            zero_bins(0, 0)

            def valid_mask(j, _p=p):
                s = j * 16 + iota
                if _p == 0:
                    return s < gtc
                return (tid * KV + s) < n_gt

            def hist_body2(j, _, _shift=shift, _src=srckey, _p=p):
                ub = _src[pl.ds(j * 16, 16)] ^ INT_MIN
                d = lax.shift_right_logical(ub, _shift) & 255
                m = valid_mask(j, _p)
                plsc.addupdate_scatter(bins, [d], one16, mask=m)
                return 0

            lax.fori_loop(0, nv, hist_body2, 0)
            publish_and_fetch_hist()

            def comb2_body(j, _):
                tot = zero16
                pre = zero16
                for t in range(NT):
                    av = allhist[pl.ds(t * 256 + j * 16, 16)]
                    tot = tot + av
                    pre = pre + jnp.where(t < tid, av, 0)
                bins[pl.ds(j * 16, 16)] = tot
                pret[pl.ds(j * 16, 16)] = pre
                return 0

            lax.fori_loop(0, 16, comb2_body, 0)

            def suff2_body(jj, run):
                j = 15 - jj
                v = bins[pl.ds(j * 16, 16)]
                rv = lax.rev(v, (0,))
                cs = plsc.cumsum(rv) + run
                suff_excl = lax.rev(cs, (0,)) - v
                offt[pl.ds(j * 16, 16)] = suff_excl + pret[pl.ds(j * 16, 16)]
                return jnp.max(cs)

            lax.fori_loop(0, 16, suff2_body, jnp.int32(0))

            def dst_body(j, _, _shift=shift, _srck=srckey, _srcd=srcdst,
                         _p=p):
                ub = _srck[pl.ds(j * 16, 16)] ^ INT_MIN
                d = lax.shift_right_logical(ub, _shift) & 255
                m = valid_mask(j, _p)
                base = plsc.load_gather(offt, [d])
                occ, last = plsc.scan_count(d, mask=m)
                occv = occ - occ_base
                _srcd[pl.ds(j * 16, 16)] = jnp.where(
                    m, base + occv, K + tid)
                plsc.store_scatter(offt, [d], base + occv + 1, mask=last)
                return 0

            lax.fori_loop(0, nv, dst_body, 0)

            if p < 3:
                pltpu.async_copy(srckey, dstkey_sh.at[srcdst], sem).wait()
                pltpu.async_copy(srcidx, dstidx_sh.at[srcdst], sem).wait()
                plsc.subcore_barrier()
                pltpu.sync_copy(dstkey_sh.at[pl.ds(tid * KV, KV)], entk)
                pltpu.sync_copy(dstidx_sh.at[pl.ds(tid * KV, KV)], enti)
            else:
                pltpu.async_copy(srcidx, perm_sh.at[srcdst], sem).wait()
                plsc.subcore_barrier()

        pltpu.sync_copy(perm_sh.at[pl.ds(tid * KV, KV)], myperm)
        row_copies = []
        for g in range(KV // 16):
            pv = myperm[pl.ds(g * 16, 16)]
            for l in range(16):
                i = g * 16 + l
                r = pv[l]
                row_copies.append(pltpu.async_copy(
                    xf_hbm.at[pl.ds(r * D, D)], rows.at[i], sem))
        for cp in row_copies:
            cp.wait()
        pltpu.sync_copy(rows, out_hbm.at[pl.ds(tid * KV, KV)])

    return k(Xf)


def kernel(X):
    return _topk_perm(X.reshape(-1))
